# baseline (speedup 1.0000x reference)
import functools

import jax
import jax.numpy as jnp
from jax import lax
from jax.experimental import pallas as pl
from jax.experimental.pallas import tpu as pltpu
from jax.experimental.pallas import tpu_sc as plsc


@functools.lru_cache(maxsize=None)
def _build(batch, length, dim):
    rows = batch * length
    info = plsc.get_sparse_core_info()
    nw = info.num_cores * info.num_subcores
    rows_per_w = rows // nw
    halves = nw // (batch * 4)
    chunk = 32
    while rows_per_w % chunk:
        chunk //= 2
    nchunk = rows_per_w // chunk

    mesh = plsc.VectorSubcoreMesh(core_axis_name="c", subcore_axis_name="s")

    @functools.partial(
        pl.kernel,
        out_type=jax.ShapeDtypeStruct((rows, dim), jnp.float32),
        mesh=mesh,
        scratch_types=[
            pltpu.VMEM((chunk, dim), jnp.float32),
            pltpu.VMEM((chunk, dim), jnp.float32),
            pltpu.SemaphoreType.DMA,
            pltpu.SemaphoreType.DMA,
        ],
    )
    def shuffle(x_hbm, out_hbm, buf0, buf1, sem0, sem1):
        bufs = (buf0, buf1)
        sems = (sem0, sem1)
        wid = lax.axis_index("s") * info.num_cores + lax.axis_index("c")
        base = wid * rows_per_w
        n = wid // (4 * halves)
        a = (wid // halves) % 4
        b0 = (wid % halves) * rows_per_w

        def gather(g, p):
            return pltpu.async_copy(
                x_hbm.at[n, pl.ds(b0 + g * chunk, chunk), a], bufs[p], sems[p]
            )

        gather(0, 0)

        def body(g, carry):
            par = g % 2
            for p in range(2):
                q = 1 - p

                @pl.when(par == p)
                def _():
                    @pl.when(g + 1 < nchunk)
                    def _():
                        gather(g + 1, q)

                    pltpu.make_async_copy(
                        x_hbm.at[n, pl.ds(b0 + g * chunk, chunk), a],
                        bufs[p],
                        sems[p],
                    ).wait()
                    pltpu.sync_copy(
                        bufs[p], out_hbm.at[pl.ds(base + g * chunk, chunk)]
                    )

            return carry

        lax.fori_loop(0, nchunk, body, 0)

    return shuffle


def kernel(inputs):
    batch, length, dim = inputs.shape
    shuffle = _build(batch, length, dim)
    out = shuffle(inputs.reshape(batch, length // 4, 4, dim))
    return out.reshape(batch, length, dim)


# speedup vs baseline: 2.0175x; 2.0175x over previous
import functools

import jax
import jax.numpy as jnp
import numpy as np
from jax import lax
from jax.experimental import pallas as pl
from jax.experimental.pallas import tpu as pltpu
from jax.experimental.pallas import tpu_sc as plsc


def _quaternary_digits(n):
    d = 1
    while n >= 4:
        n //= 4
        d += 1
    return d


def _flat_shuffle_indices(batch, length):
    digits = _quaternary_digits(length - 1)
    i = np.arange(length, dtype=np.int64)
    mask = 4**digits - 1
    idx = ((i * 4) | (i >> (2 * (digits - 1)))) & mask
    b = np.arange(batch, dtype=np.int64)[:, None]
    flat = (b * length + idx[None, :]).reshape(-1)
    return np.asarray(flat, dtype=np.int32)


def _chunk_rows(rows_per_w):
    chunk = 32
    while rows_per_w % chunk:
        chunk //= 2
    return chunk


@functools.lru_cache(maxsize=None)
def _build(batch, length, dim):
    rows = batch * length
    info = plsc.get_sparse_core_info()
    nw = info.num_cores * info.num_subcores
    rows_per_w = rows // nw
    chunk = _chunk_rows(rows_per_w)
    nchunk = rows_per_w // chunk

    mesh = plsc.VectorSubcoreMesh(core_axis_name="c", subcore_axis_name="s")

    @functools.partial(
        pl.kernel,
        out_type=jax.ShapeDtypeStruct((rows, dim), jnp.float32),
        mesh=mesh,
        scratch_types=[
            pltpu.VMEM((nchunk, chunk), jnp.int32),
            pltpu.VMEM((chunk, dim), jnp.float32),
            pltpu.VMEM((chunk, dim), jnp.float32),
            pltpu.SemaphoreType.DMA,
            pltpu.SemaphoreType.DMA,
        ],
    )
    def shuffle(x_hbm, idx_hbm, out_hbm, idx_v, buf0, buf1, sem0, sem1):
        bufs = (buf0, buf1)
        sems = (sem0, sem1)
        wid = lax.axis_index("s") * info.num_cores + lax.axis_index("c")
        base = wid * rows_per_w

        pltpu.sync_copy(idx_hbm.at[wid], idx_v)
        pltpu.async_copy(x_hbm.at[idx_v.at[0]], bufs[0], sems[0])

        def body(g, carry):
            par = g % 2
            for p in range(2):
                q = 1 - p

                @pl.when(par == p)
                def _():
                    @pl.when(g + 1 < nchunk)
                    def _():
                        pltpu.async_copy(
                            x_hbm.at[idx_v.at[g + 1]], bufs[q], sems[q]
                        )

                    pltpu.make_async_copy(
                        x_hbm.at[idx_v.at[g]], bufs[p], sems[p]
                    ).wait()
                    pltpu.sync_copy(
                        bufs[p], out_hbm.at[pl.ds(base + g * chunk, chunk)]
                    )

            return carry

        lax.fori_loop(0, nchunk, body, 0)

    return shuffle


def kernel(inputs):
    batch, length, dim = inputs.shape
    rows = batch * length
    shuffle = _build(batch, length, dim)
    info = plsc.get_sparse_core_info()
    nw = info.num_cores * info.num_subcores
    chunk = _chunk_rows(rows // nw)
    idx = jnp.asarray(_flat_shuffle_indices(batch, length)).reshape(nw, -1, chunk)
    out = shuffle(inputs.reshape(rows, dim), idx)
    return out.reshape(batch, length, dim)
